# baseline (speedup 1.0000x reference)
import contextlib
import functools

import jax
import jax.numpy as jnp
import numpy as np
from jax import lax
from jax.experimental import pallas as pl
from jax.experimental.pallas import tpu as pltpu
from jax.experimental.pallas import tpu_sc as plsc

_MASK_PROB = 0.2
_MASK_SPAN = 10

_START_CACHE = {}


def _get_starts(B, T):
    if (B, T) not in _START_CACHE:
        num_spans = max(1, int(_MASK_PROB * (T / _MASK_SPAN)))
        max_start = max(1, T - _MASK_SPAN)
        try:
            dev_ctx = jax.default_device(jax.local_devices(backend="cpu")[0])
        except Exception:
            dev_ctx = contextlib.nullcontext()
        with jax.ensure_compile_time_eval(), dev_ctx:
            keys = jax.random.split(jax.random.key(42), B)
            rows = [np.asarray(jax.random.permutation(k, max_start))[:num_spans]
                    for k in keys]
        _START_CACHE[(B, T)] = np.stack(rows).astype(np.int32)
    return _START_CACHE[(B, T)]


_NC, _NS = 2, 16
_NW = _NC * _NS
_CH = 64
_IDX_CH = 32


def _masked_row_table(B, T, starts):
    rows_per_w = (B * T) // _NW
    mask = np.zeros((B, T), dtype=bool)
    for b in range(B):
        for s in starts[b]:
            mask[b, s:min(T, s + _MASK_SPAN)] = True
    flat = np.flatnonzero(mask.reshape(-1)).astype(np.int32)
    per_w = [flat[(flat >= w * rows_per_w) & (flat < (w + 1) * rows_per_w)]
             for w in range(_NW)]
    maxn = max(len(p) for p in per_w)
    assert min(len(p) for p in per_w) > 0
    nch = -(-maxn // _IDX_CH)
    tab = np.zeros((_NW, nch, _IDX_CH), dtype=np.int32)
    for w, p in enumerate(per_w):
        padded = np.concatenate([p, np.full(nch * _IDX_CH - len(p), p[-1],
                                            dtype=np.int32)])
        tab[w] = padded.reshape(nch, _IDX_CH)
    return tab


def _sc_body(rows_per_w, nch, z_ref, idx_ref, emb_ref, out_ref,
             buf, emb_rep, idx_v, sem):
    wid = lax.axis_index("s") * _NC + lax.axis_index("c")
    base = wid * rows_per_w
    for j in range(_IDX_CH):
        pltpu.sync_copy(emb_ref, emb_rep.at[j])
    for i in range(rows_per_w // _CH):
        pltpu.sync_copy(z_ref.at[pl.ds(base + i * _CH, _CH)], buf)
        pltpu.sync_copy(buf, out_ref.at[pl.ds(base + i * _CH, _CH)])
    pltpu.sync_copy(idx_ref.at[wid], idx_v)
    for j in range(nch):
        pltpu.async_copy(emb_rep, out_ref.at[idx_v.at[j]], sem).wait()


def _mask_body(span, starts_col_ref, m_ref):
    b = pl.program_id(0)
    del b
    t_full = m_ref.shape[2]
    s_padc = starts_col_ref.shape[1]
    stc = starts_col_ref[0]
    tic = jax.lax.broadcasted_iota(jnp.int32, (s_padc, t_full), 1)
    hitc = (tic >= stc) & (tic < stc + span)
    m_ref[0] = jnp.any(hitc, axis=0, keepdims=True)


def kernel(z_t, mask_emb):
    B, T, D = z_t.shape
    starts = _get_starts(B, T)
    S = starts.shape[1]
    rows_per_w = (B * T) // _NW
    tab = _masked_row_table(B, T, starts)
    nch = tab.shape[1]
    tab = jnp.asarray(tab)

    mesh = plsc.VectorSubcoreMesh(core_axis_name="c", subcore_axis_name="s")
    sc = pl.kernel(
        functools.partial(_sc_body, rows_per_w, nch),
        out_type=jax.ShapeDtypeStruct((B * T, D), z_t.dtype),
        mesh=mesh,
        scratch_types=[
            pltpu.VMEM((_CH, D), jnp.float32),
            pltpu.VMEM((_IDX_CH, D), jnp.float32),
            pltpu.VMEM((nch, _IDX_CH), jnp.int32),
            pltpu.SemaphoreType.DMA,
        ],
    )
    out = sc(z_t.reshape(B * T, D), tab, mask_emb)

    S_PADC = -(-S // 8) * 8
    starts_col = np.full((B, S_PADC, 1), T, dtype=np.int32)
    starts_col[:, :S, 0] = starts
    starts_col = jnp.asarray(starts_col)
    mask = pl.pallas_call(
        functools.partial(_mask_body, _MASK_SPAN),
        grid=(B,),
        in_specs=[pl.BlockSpec((1, S_PADC, 1), lambda b: (b, 0, 0))],
        out_specs=pl.BlockSpec((1, 1, T), lambda b: (b, 0, 0)),
        out_shape=jax.ShapeDtypeStruct((B, 1, T), jnp.bool_),
    )(starts_col)
    return out.reshape(B, T, D), mask.reshape(B, T)


# speedup vs baseline: 1.0442x; 1.0442x over previous
import contextlib
import functools

import jax
import jax.numpy as jnp
import numpy as np
from jax import lax
from jax.experimental import pallas as pl
from jax.experimental.pallas import tpu as pltpu
from jax.experimental.pallas import tpu_sc as plsc

_MASK_PROB = 0.2
_MASK_SPAN = 10

_START_CACHE = {}


def _get_starts(B, T):
    if (B, T) not in _START_CACHE:
        num_spans = max(1, int(_MASK_PROB * (T / _MASK_SPAN)))
        max_start = max(1, T - _MASK_SPAN)
        try:
            dev_ctx = jax.default_device(jax.local_devices(backend="cpu")[0])
        except Exception:
            dev_ctx = contextlib.nullcontext()
        with jax.ensure_compile_time_eval(), dev_ctx:
            keys = jax.random.split(jax.random.key(42), B)
            rows = [np.asarray(jax.random.permutation(k, max_start))[:num_spans]
                    for k in keys]
        _START_CACHE[(B, T)] = np.stack(rows).astype(np.int32)
    return _START_CACHE[(B, T)]


_NC, _NS = 2, 16
_NW = _NC * _NS
_CH = 32
_IDX_CH = 32


def _masked_row_table(B, T, starts):
    rows_per_w = (B * T) // _NW
    mask = np.zeros((B, T), dtype=bool)
    for b in range(B):
        for s in starts[b]:
            mask[b, s:min(T, s + _MASK_SPAN)] = True
    flat = np.flatnonzero(mask.reshape(-1)).astype(np.int32)
    per_w = [flat[(flat >= w * rows_per_w) & (flat < (w + 1) * rows_per_w)]
             for w in range(_NW)]
    maxn = max(len(p) for p in per_w)
    assert min(len(p) for p in per_w) > 0
    nch = -(-maxn // _IDX_CH)
    tab = np.zeros((_NW, nch, _IDX_CH), dtype=np.int32)
    for w, p in enumerate(per_w):
        padded = np.concatenate([p, np.full(nch * _IDX_CH - len(p), p[-1],
                                            dtype=np.int32)])
        tab[w] = padded.reshape(nch, _IDX_CH)
    return tab


def _sc_body(rows_per_w, nch, z_ref, idx_ref, emb_ref, out_ref,
             buf0, buf1, emb_rep, idx_v, gsem, ssem):
    wid = lax.axis_index("s") * _NC + lax.axis_index("c")
    base = wid * rows_per_w
    for j in range(_IDX_CH):
        pltpu.sync_copy(emb_ref, emb_rep.at[j])
    pltpu.sync_copy(idx_ref.at[wid], idx_v)
    bufs = (buf0, buf1)
    n = rows_per_w // _CH
    gd = {}
    sd = {}
    gd[0] = pltpu.make_async_copy(z_ref.at[pl.ds(base, _CH)], bufs[0], gsem)
    gd[0].start()
    for i in range(n):
        gd[i].wait()
        sd[i] = pltpu.make_async_copy(
            bufs[i % 2], out_ref.at[pl.ds(base + i * _CH, _CH)], ssem)
        sd[i].start()
        if i + 1 < n:
            if i >= 1:
                sd[i - 1].wait()
            gd[i + 1] = pltpu.make_async_copy(
                z_ref.at[pl.ds(base + (i + 1) * _CH, _CH)],
                bufs[(i + 1) % 2], gsem)
            gd[i + 1].start()
    sd[n - 2].wait()
    sd[n - 1].wait()
    for j in range(nch):
        pltpu.async_copy(emb_rep, out_ref.at[idx_v.at[j]], gsem).wait()


def _mask_body(span, starts_col_ref, m_ref):
    b = pl.program_id(0)
    del b
    t_full = m_ref.shape[2]
    s_padc = starts_col_ref.shape[1]
    stc = starts_col_ref[0]
    tic = jax.lax.broadcasted_iota(jnp.int32, (s_padc, t_full), 1)
    hitc = (tic >= stc) & (tic < stc + span)
    m_ref[0] = jnp.any(hitc, axis=0, keepdims=True)


def kernel(z_t, mask_emb):
    B, T, D = z_t.shape
    starts = _get_starts(B, T)
    S = starts.shape[1]
    rows_per_w = (B * T) // _NW
    tab = _masked_row_table(B, T, starts)
    nch = tab.shape[1]
    tab = jnp.asarray(tab)

    mesh = plsc.VectorSubcoreMesh(core_axis_name="c", subcore_axis_name="s")
    sc = pl.kernel(
        functools.partial(_sc_body, rows_per_w, nch),
        out_type=jax.ShapeDtypeStruct((B * T, D), z_t.dtype),
        mesh=mesh,
        scratch_types=[
            pltpu.VMEM((_CH, D), jnp.float32),
            pltpu.VMEM((_CH, D), jnp.float32),
            pltpu.VMEM((_IDX_CH, D), jnp.float32),
            pltpu.VMEM((nch, _IDX_CH), jnp.int32),
            pltpu.SemaphoreType.DMA,
            pltpu.SemaphoreType.DMA,
        ],
    )
    out = sc(z_t.reshape(B * T, D), tab, mask_emb)

    S_PADC = -(-S // 8) * 8
    starts_col = np.full((B, S_PADC, 1), T, dtype=np.int32)
    starts_col[:, :S, 0] = starts
    starts_col = jnp.asarray(starts_col)
    mask = pl.pallas_call(
        functools.partial(_mask_body, _MASK_SPAN),
        grid=(B,),
        in_specs=[pl.BlockSpec((1, S_PADC, 1), lambda b: (b, 0, 0))],
        out_specs=pl.BlockSpec((1, 1, T), lambda b: (b, 0, 0)),
        out_shape=jax.ShapeDtypeStruct((B, 1, T), jnp.bool_),
    )(starts_col)
    return out.reshape(B, T, D), mask.reshape(B, T)
